# baseline (speedup 1.0000x reference)
import jax
import jax.numpy as jnp
from jax import lax
from jax.experimental import pallas as pl
from jax.experimental.pallas import tpu as pltpu
from jax.experimental.pallas import tpu_sc as plsc

B = 4
T = 8192
V = 128
NSC = 7
NEC = 8
EXP = 1 + NSC + NEC
ROW = V * EXP
ST = V * NSC
EN = V * NEC
BC = ST + EN
L = 16
NW = 32
TPW = T // NW


def _tec_body(inp_hbm, enc_hbm, st_hbm, out_hbm, bc_v, in_v, out_v, idx_v, idxi_v):
    wid = lax.axis_index("s") * 2 + lax.axis_index("c")
    lanes = lax.iota(jnp.int32, 16)

    @pl.loop(0, ST // L)
    def _(r):
        s = r * L + lanes
        v = (s * 9363) >> 16
        idx_v[pl.ds(r * L, L)] = v * 16 + (s - v * 7) + 1

    @pl.loop(0, EN // L)
    def _(r):
        e = r * L + lanes
        idx_v[pl.ds(ST + r * L, L)] = ((e >> 3) << 4) + 8 + (e & 7)

    @pl.loop(0, V // L)
    def _(r):
        v = r * L + lanes
        idxi_v[pl.ds(r * L, L)] = v << 4

    t0 = wid * TPW

    @pl.loop(0, TPW)
    def _(i):
        t = t0 + i
        pltpu.sync_copy(st_hbm.at[t], bc_v.at[pl.ds(0, ST)])
        pltpu.sync_copy(enc_hbm.at[t], bc_v.at[pl.ds(ST, EN)])
        for b in range(B):
            pltpu.sync_copy(inp_hbm.at[b, t], in_v.at[pl.ds(b * V, V)])

        @pl.loop(0, BC // L)
        def _(r):
            x = bc_v[pl.ds(r * L, L)]
            d = idx_v[pl.ds(r * L, L)]
            for b in range(B):
                plsc.store_scatter(out_v, [d + (b * ROW)], x)

        @pl.loop(0, V // L)
        def _(r):
            d = idxi_v[pl.ds(r * L, L)]
            for b in range(B):
                x = in_v[pl.ds(b * V + r * L, L)]
                plsc.store_scatter(out_v, [d + (b * ROW)], x)

        for b in range(B):
            pltpu.sync_copy(out_v.at[pl.ds(b * ROW, ROW)], out_hbm.at[b, t])


@jax.jit
def kernel(inp, variable_encodings, static_channels):
    run = pl.kernel(
        _tec_body,
        out_type=jax.ShapeDtypeStruct((B, T, ROW), jnp.float32),
        mesh=plsc.VectorSubcoreMesh(core_axis_name="c", subcore_axis_name="s"),
        compiler_params=pltpu.CompilerParams(needs_layout_passes=False),
        scratch_types=[
            pltpu.VMEM((BC,), jnp.float32),
            pltpu.VMEM((B * V,), jnp.float32),
            pltpu.VMEM((B * ROW,), jnp.float32),
            pltpu.VMEM((BC,), jnp.int32),
            pltpu.VMEM((V,), jnp.int32),
        ],
    )
    return run(inp, variable_encodings, static_channels)


# speedup vs baseline: 3.3508x; 3.3508x over previous
import jax
import jax.numpy as jnp
from jax import lax
from jax.experimental import pallas as pl
from jax.experimental.pallas import tpu as pltpu
from jax.experimental.pallas import tpu_sc as plsc

B = 4
T = 8192
V = 128
NSC = 7
NEC = 8
EXP = 1 + NSC + NEC
ROW = V * EXP
ST = V * NSC
EN = V * NEC
BC = ST + EN
L = 16
NW = 32
TPW = T // NW


def _tec_body(inp_hbm, enc_hbm, st_hbm, out_hbm,
              bc_v, in_v, out_v, idx_v, idxi_v,
              sin0, sin1, sout0, sout1):
    wid = lax.axis_index("s") * 2 + lax.axis_index("c")
    lanes = lax.iota(jnp.int32, 16)
    sin = (sin0, sin1)
    sout = (sout0, sout1)

    @pl.loop(0, ST // L)
    def _(r):
        s = r * L + lanes
        v = (s * 9363) >> 16
        idx_v[pl.ds(r * L, L)] = v * 16 + (s - v * 7) + 1

    @pl.loop(0, EN // L)
    def _(r):
        e = r * L + lanes
        idx_v[pl.ds(ST + r * L, L)] = ((e >> 3) << 4) + 8 + (e & 7)

    @pl.loop(0, V // L)
    def _(r):
        v = r * L + lanes
        idxi_v[pl.ds(r * L, L)] = v << 4

    def start_in(k, t):
        pltpu.async_copy(st_hbm.at[t], bc_v.at[k, pl.ds(0, ST)], sin[k])
        pltpu.async_copy(enc_hbm.at[t], bc_v.at[k, pl.ds(ST, EN)], sin[k])
        pltpu.async_copy(inp_hbm.at[:, t], in_v.at[k], sin[k])

    def wait_in(k):
        pltpu.make_async_copy(st_hbm.at[0], bc_v.at[k, pl.ds(0, ST)], sin[k]).wait()
        pltpu.make_async_copy(enc_hbm.at[0], bc_v.at[k, pl.ds(ST, EN)], sin[k]).wait()
        pltpu.make_async_copy(inp_hbm.at[:, 0], in_v.at[k], sin[k]).wait()

    def start_out(k, t):
        pltpu.async_copy(out_v.at[k], out_hbm.at[:, t], sout[k])

    def wait_out(k):
        pltpu.make_async_copy(out_v.at[k], out_hbm.at[:, 0], sout[k]).wait()

    def build(k):
        @pl.loop(0, BC // L)
        def _(r):
            x = bc_v[k, pl.ds(r * L, L)]
            d = idx_v[pl.ds(r * L, L)]
            for b in range(B):
                rb = jnp.full((L,), b, jnp.int32)
                plsc.store_scatter(out_v.at[k], [rb, d], x)

        @pl.loop(0, V // L)
        def _(r):
            d = idxi_v[pl.ds(r * L, L)]
            for b in range(B):
                x = in_v[k, b, pl.ds(r * L, L)]
                rb = jnp.full((L,), b, jnp.int32)
                plsc.store_scatter(out_v.at[k], [rb, d], x)

    t0 = wid * TPW
    start_in(0, t0)
    start_in(1, t0 + 1)
    for k in range(2):
        wait_in(k)
        build(k)
        start_out(k, t0 + k)
        start_in(k, t0 + k + 2)

    @pl.loop(0, (TPW - 4) // 2)
    def _(ii):
        i = 2 + ii * 2
        for k in range(2):
            t = t0 + i + k
            wait_in(k)
            wait_out(k)
            build(k)
            start_out(k, t)
            start_in(k, t + 2)

    for k in range(2):
        wait_in(k)
        wait_out(k)
        build(k)
        start_out(k, t0 + TPW - 2 + k)
    for k in range(2):
        wait_out(k)


@jax.jit
def kernel(inp, variable_encodings, static_channels):
    run = pl.kernel(
        _tec_body,
        out_type=jax.ShapeDtypeStruct((B, T, ROW), jnp.float32),
        mesh=plsc.VectorSubcoreMesh(core_axis_name="c", subcore_axis_name="s"),
        compiler_params=pltpu.CompilerParams(needs_layout_passes=False),
        scratch_types=[
            pltpu.VMEM((2, BC), jnp.float32),
            pltpu.VMEM((2, B, V), jnp.float32),
            pltpu.VMEM((2, B, ROW), jnp.float32),
            pltpu.VMEM((BC,), jnp.int32),
            pltpu.VMEM((V,), jnp.int32),
            pltpu.SemaphoreType.DMA,
            pltpu.SemaphoreType.DMA,
            pltpu.SemaphoreType.DMA,
            pltpu.SemaphoreType.DMA,
        ],
    )
    return run(inp, variable_encodings, static_channels)
